# baseline (speedup 1.0000x reference)
import functools

import jax
import jax.numpy as jnp
from jax import lax
from jax.experimental import pallas as pl
from jax.experimental.pallas import tpu as pltpu
from jax.experimental.pallas import tpu_sc as plsc

N = 10000
E = 320000
F = 128

N_PAD = 10112
RPT = N_PAD // 16

TILES = 32
CW = 32
EPT = 10240
CH0 = 160
CH1 = 480
CHMAX = max(CH0, CH1)
TOTCH = 16 * (CH0 + CH1)
EPAD = TOTCH * CW

_mesh = plsc.VectorSubcoreMesh(core_axis_name="c", subcore_axis_name="s")
_sc_params = pltpu.CompilerParams(needs_layout_passes=False,
                                  use_tc_tiling_on_sc=False)


@functools.partial(
    pl.kernel,
    mesh=_mesh,
    compiler_params=_sc_params,
    out_type=jax.ShapeDtypeStruct((TILES, N_PAD), jnp.float32),
    scratch_types=[
        pltpu.VMEM((EPT // 16, 16), jnp.int32),
        pltpu.VMEM((N_PAD,), jnp.float32),
    ],
)
def _deg_kernel(dst_hbm, out_hbm, idx_v, deg_v):
    cid = lax.axis_index("c")
    sid = lax.axis_index("s")
    t = cid * 16 + sid

    pltpu.sync_copy(dst_hbm.at[t], idx_v)

    def _zero(r, _):
        deg_v[pl.ds(r * 16, 16)] = jnp.zeros((16,), jnp.float32)
        return _
    lax.fori_loop(0, N_PAD // 16, _zero, None)

    def _scat(j, _):
        ones = jnp.full((16,), 1.0, jnp.float32)
        v = idx_v[j, :]
        plsc.addupdate_scatter(deg_v, [v], ones)
        return _
    lax.fori_loop(0, EPT // 16, _scat, None)

    pltpu.sync_copy(deg_v, out_hbm.at[t])


def _lin_body(dp_ref, x_ref, w_ref, y_ref, y2_ref, dinv_ref):
    deg = jnp.sum(dp_ref[...], axis=1, keepdims=True) + 1.0
    dinv = lax.rsqrt(deg)
    xw = jnp.dot(x_ref[...], w_ref[...], preferred_element_type=jnp.float32)
    y = xw * dinv
    y_ref[...] = y
    y2_ref[0] = y
    y2_ref[1] = y
    dinv_ref[...] = dinv


_lin_call = pl.pallas_call(
    _lin_body,
    out_shape=(
        jax.ShapeDtypeStruct((N_PAD, F), jnp.float32),
        jax.ShapeDtypeStruct((2, N_PAD, F), jnp.float32),
        jax.ShapeDtypeStruct((N_PAD, 1), jnp.float32),
    ),
)


@functools.partial(
    pl.kernel,
    mesh=_mesh,
    compiler_params=_sc_params,
    out_type=jax.ShapeDtypeStruct((2, N_PAD, F), jnp.float32),
    scratch_types=[
        pltpu.VMEM((CHMAX, CW), jnp.int32),
        pltpu.VMEM((CHMAX, CW), jnp.int32),
        pltpu.VMEM((4, CW, F), jnp.float32),
        pltpu.VMEM_SHARED((N_PAD, F), jnp.float32),
        [pltpu.SemaphoreType.DMA] * 4,
        [pltpu.SemaphoreType.DMA] * 4,
    ],
)
def _agg_kernel(y_hbm, src_hbm, dst_hbm, out_hbm, idx_s, idx_d, rows, acc,
                sg, ss):
    cid = lax.axis_index("c")
    sid = lax.axis_index("s")

    base_ch = jnp.where(cid == 0, sid * CH0, 16 * CH0 + sid * CH1)
    my_ch = jnp.where(cid == 0, CH0, CH1)
    pltpu.sync_copy(src_hbm.at[pl.ds(base_ch, CHMAX)], idx_s)
    pltpu.sync_copy(dst_hbm.at[pl.ds(base_ch, CHMAX)], idx_d)

    def _zero(r, _):
        for k in range(F // 16):
            rows[0, r, pl.ds(k * 16, 16)] = jnp.zeros((16,), jnp.float32)
        return _
    lax.fori_loop(0, CW, _zero, None)

    base = sid * RPT
    nfull, rem = divmod(RPT, CW)
    for q in range(nfull):
        pltpu.sync_copy(rows.at[0], acc.at[pl.ds(base + q * CW, CW)])
    if rem:
        pltpu.sync_copy(rows.at[0, pl.ds(0, rem)],
                        acc.at[pl.ds(base + nfull * CW, rem)])
    plsc.subcore_barrier()

    def _gath(j, b):
        return pltpu.async_copy(y_hbm.at[cid].at[idx_s.at[j]], rows.at[b],
                                sg[b])

    def _gath_wait(j, b):
        pltpu.make_async_copy(y_hbm.at[cid].at[idx_s.at[j]], rows.at[b],
                              sg[b]).wait()

    def _scat(j, b):
        return pltpu.async_copy(rows.at[b], acc.at[idx_d.at[j]], ss[b],
                                add=True)

    def _scat_wait(j, b):
        pltpu.make_async_copy(rows.at[b], acc.at[idx_d.at[j]], ss[b]).wait()

    _gath(0, 0)
    _gath(1, 1)

    def _quad(k, _):
        j0 = 4 * k
        for u in range(4):
            j = j0 + u
            b = u
            bg = (u + 2) % 4

            @pl.when(jnp.logical_and(j >= 2, j + 2 < my_ch))
            def _():
                _scat_wait(j - 2, bg)
                _gath(j + 2, bg)

            @pl.when(jnp.logical_and(j < 2, j + 2 < my_ch))
            def _():
                _gath(j + 2, bg)

            _gath_wait(j, b)
            _scat(j, b)
        return _
    lax.fori_loop(0, my_ch // 4, _quad, None)
    for u in range(4):
        _scat_wait(my_ch - 4 + u, u)

    plsc.subcore_barrier()
    for q in range(nfull):
        pltpu.sync_copy(acc.at[pl.ds(base + q * CW, CW)], rows.at[0])
        pltpu.sync_copy(rows.at[0], out_hbm.at[cid, pl.ds(base + q * CW, CW)])
    pltpu.sync_copy(acc.at[pl.ds(base + nfull * CW, rem)],
                    rows.at[0, pl.ds(0, rem)])
    pltpu.sync_copy(rows.at[0, pl.ds(0, rem)],
                    out_hbm.at[cid, pl.ds(base + nfull * CW, rem)])


def _fin_body(a_ref, y_ref, dinv_ref, b_ref, o_ref):
    s = (a_ref[0] + a_ref[1] + y_ref[...]) * dinv_ref[...] + b_ref[...]
    o_ref[...] = jnp.maximum(s, 0.0)


_fin_call = pl.pallas_call(
    _fin_body,
    out_shape=jax.ShapeDtypeStruct((N_PAD, F), jnp.float32),
)


def kernel(x, edge_index, W, b):
    src = edge_index[0].astype(jnp.int32)
    dst = edge_index[1].astype(jnp.int32)
    pad = EPAD - E
    src_p = jnp.concatenate([src, jnp.full((pad,), N, jnp.int32)]).reshape(TOTCH, CW)
    dst_p = jnp.concatenate([dst, jnp.full((pad,), N, jnp.int32)]).reshape(TOTCH, CW)
    x_p = jnp.pad(x, ((0, N_PAD - N), (0, 0)))

    dst_a = dst_p.reshape(TILES, EPT // 16, 16)
    dp = _deg_kernel(dst_a)
    y, y2, dinv = _lin_call(dp.T, x_p, W)
    agg = _agg_kernel(y2, src_p, dst_p)
    out = _fin_call(agg, y, dinv, b.reshape(1, F))
    return out[:N]


# speedup vs baseline: 1.0436x; 1.0436x over previous
import functools

import jax
import jax.numpy as jnp
from jax import lax
from jax.experimental import pallas as pl
from jax.experimental.pallas import tpu as pltpu
from jax.experimental.pallas import tpu_sc as plsc

N = 10000
E = 320000
F = 128

N_PAD = 10112
RPT = N_PAD // 16

TILES = 32
CW = 40
CH = 256
EPT = CH * CW
EPAD = TILES * EPT

_mesh = plsc.VectorSubcoreMesh(core_axis_name="c", subcore_axis_name="s")
_sc_params = pltpu.CompilerParams(needs_layout_passes=False,
                                  use_tc_tiling_on_sc=False)


@functools.partial(
    pl.kernel,
    mesh=_mesh,
    compiler_params=_sc_params,
    out_type=jax.ShapeDtypeStruct((TILES, N_PAD), jnp.float32),
    scratch_types=[
        pltpu.VMEM((EPT // 16, 16), jnp.int32),
        pltpu.VMEM((N_PAD,), jnp.float32),
    ],
)
def _deg_kernel(dst_hbm, out_hbm, idx_v, deg_v):
    cid = lax.axis_index("c")
    sid = lax.axis_index("s")
    t = cid * 16 + sid

    pltpu.sync_copy(dst_hbm.at[t], idx_v)

    def _zero(r, _):
        deg_v[pl.ds(r * 16, 16)] = jnp.zeros((16,), jnp.float32)
        return _
    lax.fori_loop(0, N_PAD // 16, _zero, None)

    def _scat(j, _):
        ones = jnp.full((16,), 1.0, jnp.float32)
        v = idx_v[j, :]
        plsc.addupdate_scatter(deg_v, [v], ones)
        return _
    lax.fori_loop(0, EPT // 16, _scat, None)

    pltpu.sync_copy(deg_v, out_hbm.at[t])


def _lin_body(dp_ref, x_ref, w_ref, y_ref, y2_ref, dinv_ref):
    deg = jnp.sum(dp_ref[...], axis=1, keepdims=True) + 1.0
    dinv = lax.rsqrt(deg)
    xw = jnp.dot(x_ref[...], w_ref[...], preferred_element_type=jnp.float32)
    y = xw * dinv
    y_ref[...] = y
    y2_ref[0] = y
    y2_ref[1] = y
    dinv_ref[...] = dinv


_lin_call = pl.pallas_call(
    _lin_body,
    out_shape=(
        jax.ShapeDtypeStruct((N_PAD, F), jnp.float32),
        jax.ShapeDtypeStruct((2, N_PAD, F), jnp.float32),
        jax.ShapeDtypeStruct((N_PAD, 1), jnp.float32),
    ),
)


@functools.partial(
    pl.kernel,
    mesh=_mesh,
    compiler_params=_sc_params,
    out_type=jax.ShapeDtypeStruct((2, N_PAD, F), jnp.float32),
    scratch_types=[
        pltpu.VMEM((CH, CW), jnp.int32),
        pltpu.VMEM((CH, CW), jnp.int32),
        pltpu.VMEM((4, CW, F), jnp.float32),
        pltpu.VMEM_SHARED((N_PAD, F), jnp.float32),
        [pltpu.SemaphoreType.DMA] * 4,
        [pltpu.SemaphoreType.DMA] * 4,
    ],
)
def _agg_kernel(y_hbm, src_hbm, dst_hbm, out_hbm, idx_s, idx_d, rows, acc,
                sg, ss):
    cid = lax.axis_index("c")
    sid = lax.axis_index("s")
    t = cid * 16 + sid

    pltpu.sync_copy(src_hbm.at[t], idx_s)
    pltpu.sync_copy(dst_hbm.at[t], idx_d)

    def _zero(r, _):
        for k in range(F // 16):
            rows[0, r, pl.ds(k * 16, 16)] = jnp.zeros((16,), jnp.float32)
        return _
    lax.fori_loop(0, CW, _zero, None)

    base = sid * RPT
    nfull, rem = divmod(RPT, CW)
    for q in range(nfull):
        pltpu.sync_copy(rows.at[0], acc.at[pl.ds(base + q * CW, CW)])
    if rem:
        pltpu.sync_copy(rows.at[0, pl.ds(0, rem)],
                        acc.at[pl.ds(base + nfull * CW, rem)])
    plsc.subcore_barrier()

    def _gath(j, b):
        return pltpu.async_copy(y_hbm.at[cid].at[idx_s.at[j]], rows.at[b],
                                sg[b])

    def _gath_wait(j, b):
        pltpu.make_async_copy(y_hbm.at[cid].at[idx_s.at[j]], rows.at[b],
                              sg[b]).wait()

    def _scat(j, b):
        return pltpu.async_copy(rows.at[b], acc.at[idx_d.at[j]], ss[b],
                                add=True)

    def _scat_wait(j, b):
        pltpu.make_async_copy(rows.at[b], acc.at[idx_d.at[j]], ss[b]).wait()

    _gath(0, 0)
    _gath(1, 1)

    def _quad(k, _):
        j0 = 4 * k
        for u in range(4):
            j = j0 + u
            b = u
            bg = (u + 2) % 4

            @pl.when(jnp.logical_and(j >= 2, j + 2 < CH))
            def _():
                _scat_wait(j - 2, bg)
                _gath(j + 2, bg)

            @pl.when(jnp.logical_and(j < 2, j + 2 < CH))
            def _():
                _gath(j + 2, bg)

            _gath_wait(j, b)
            _scat(j, b)
        return _
    lax.fori_loop(0, CH // 4, _quad, None)
    for u in range(4):
        _scat_wait(CH - 4 + u, u)

    plsc.subcore_barrier()
    for q in range(nfull):
        pltpu.sync_copy(acc.at[pl.ds(base + q * CW, CW)], rows.at[0])
        pltpu.sync_copy(rows.at[0], out_hbm.at[cid, pl.ds(base + q * CW, CW)])
    pltpu.sync_copy(acc.at[pl.ds(base + nfull * CW, rem)],
                    rows.at[0, pl.ds(0, rem)])
    pltpu.sync_copy(rows.at[0, pl.ds(0, rem)],
                    out_hbm.at[cid, pl.ds(base + nfull * CW, rem)])


def _fin_body(a_ref, y_ref, dinv_ref, b_ref, o_ref):
    s = (a_ref[0] + a_ref[1] + y_ref[...]) * dinv_ref[...] + b_ref[...]
    o_ref[...] = jnp.maximum(s, 0.0)


_fin_call = pl.pallas_call(
    _fin_body,
    out_shape=jax.ShapeDtypeStruct((N_PAD, F), jnp.float32),
)


def kernel(x, edge_index, W, b):
    src = edge_index[0].astype(jnp.int32)
    dst = edge_index[1].astype(jnp.int32)
    pad = EPAD - E
    src_p = jnp.concatenate([src, jnp.full((pad,), N, jnp.int32)]).reshape(TILES, CH, CW)
    dst_p = jnp.concatenate([dst, jnp.full((pad,), N, jnp.int32)]).reshape(TILES, CH, CW)
    x_p = jnp.pad(x, ((0, N_PAD - N), (0, 0)))

    dst_a = dst_p.reshape(TILES, EPT // 16, 16)
    dp = _deg_kernel(dst_a)
    y, y2, dinv = _lin_call(dp.T, x_p, W)
    agg = _agg_kernel(y2, src_p, dst_p)
    out = _fin_call(agg, y, dinv, b.reshape(1, F))
    return out[:N]
